# baseline (speedup 1.0000x reference)
import jax
import jax.numpy as jnp
from jax.experimental import pallas as pl

_NSTREAM = 2
_BV = 2048


def _mm_kernel(x_ref, *refs):
    w_refs = refs[:_NSTREAM]
    o_ref = refs[_NSTREAM]
    x = x_ref[:, 0, :]
    for k in range(_NSTREAM):
        o_ref[:, 0, k * _BV:(k + 1) * _BV] = jax.lax.dot_general(
            x, w_refs[k][...],
            dimension_numbers=(((1,), (1,)), ((), ())),
            preferred_element_type=jnp.float32,
        )


def kernel(hidden_states, W):
    B, Q, H = hidden_states.shape
    V = W.shape[0]
    step = _NSTREAM * _BV
    last_valid = (V - 1) // _BV
    w_specs = [
        pl.BlockSpec(
            (_BV, H),
            lambda i, k=k: (jnp.minimum(i * _NSTREAM + k, last_valid), 0),
        )
        for k in range(_NSTREAM)
    ]
    return pl.pallas_call(
        _mm_kernel,
        grid=(pl.cdiv(V, step),),
        in_specs=[pl.BlockSpec((B, Q, H), lambda i: (0, 0, 0))] + w_specs,
        out_specs=pl.BlockSpec((B, Q, step), lambda i: (0, 0, i)),
        out_shape=jax.ShapeDtypeStruct((B, Q, V), jnp.float32),
    )(hidden_states, *([W] * _NSTREAM))


# speedup vs baseline: 1.0169x; 1.0169x over previous
import jax
import jax.numpy as jnp
from jax.experimental import pallas as pl

_BV = 4096


def _mm_kernel(x_ref, w_ref, o_ref):
    o_ref[:, 0, :] = jax.lax.dot_general(
        x_ref[:, 0, :], w_ref[...],
        dimension_numbers=(((1,), (1,)), ((), ())),
        preferred_element_type=jnp.float32,
    )


def kernel(hidden_states, W):
    B, Q, H = hidden_states.shape
    V = W.shape[0]
    last_valid = (V - 1) // _BV
    return pl.pallas_call(
        _mm_kernel,
        grid=(pl.cdiv(V, _BV),),
        in_specs=[
            pl.BlockSpec((B, Q, H), lambda i: (0, 0, 0)),
            pl.BlockSpec((_BV, H), lambda i: (jnp.minimum(i, last_valid), 0)),
        ],
        out_specs=pl.BlockSpec((B, Q, _BV), lambda i: (0, 0, i)),
        out_shape=jax.ShapeDtypeStruct((B, Q, V), jnp.float32),
    )(hidden_states, W)
